# baseline (speedup 1.0000x reference)
import functools

import jax
import jax.numpy as jnp
from jax import lax
from jax.experimental import pallas as pl
from jax.experimental.pallas import tpu as pltpu
from jax.experimental.pallas import tpu_sc as plsc

_B = 16
_N = 32768
_NS = 16
_L = 16
_C = _N // _NS
_V = _C // _L

_mesh = plsc.VectorSubcoreMesh(core_axis_name="c", subcore_axis_name="s")


@functools.partial(
    pl.kernel,
    mesh=_mesh,
    out_type=jax.ShapeDtypeStruct((_B,), jnp.float32),
    scratch_types=[
        pltpu.VMEM((_C,), jnp.float32),
        pltpu.VMEM((_C,), jnp.float32),
        pltpu.VMEM((_C,), jnp.int32),
        pltpu.VMEM((_C,), jnp.float32),
        pltpu.VMEM((_B,), jnp.float32),
        pltpu.VMEM((_NS * _B,), jnp.float32),
        pltpu.VMEM_SHARED((_NS * _B,), jnp.float32),
        pltpu.SemaphoreType.DMA,
    ],
)
def _segsum_sc(scores_hbm, gating_hbm, ids_hbm, out_hbm,
               s_v, g_v, i_v, p_v, part_v, all_v, acc_sh, sem):
    cid = lax.axis_index("c")
    sid = lax.axis_index("s")

    @pl.when(cid == 0)
    def _():
        base = sid * _C

        c1 = pltpu.async_copy(scores_hbm.at[pl.ds(base, _C)], s_v, sem)
        c2 = pltpu.async_copy(gating_hbm.at[pl.ds(base, _C)], g_v, sem)
        c3 = pltpu.async_copy(ids_hbm.at[pl.ds(base, _C)], i_v, sem)
        c1.wait()
        c2.wait()
        c3.wait()

        first = i_v[pl.ds(0, _L)][0]
        last = i_v[pl.ds(_C - _L, _L)][_L - 1]
        lanes = lax.iota(jnp.int32, _L)

        def lane_sum(acc):
            half = [acc[2 * l] + acc[2 * l + 1] for l in range(_L // 2)]
            while len(half) > 1:
                half = [half[2 * l] + half[2 * l + 1]
                        for l in range(len(half) // 2)]
            return half[0]

        @pl.when(first == last)
        def _():
            accs4 = [jnp.zeros((_L,), jnp.float32) for _ in range(4)]
            for j in range(_V):
                sl = pl.ds(j * _L, _L)
                accs4[j % 4] = accs4[j % 4] + s_v[sl] * g_v[sl]
            acc = (accs4[0] + accs4[1]) + (accs4[2] + accs4[3])
            part_v[...] = jnp.where(lanes == first, lane_sum(acc), 0.0)

        @pl.when(first != last)
        def _():
            def pbody(j, _):
                sl = pl.ds(j * _L, _L)
                p_v[sl] = s_v[sl] * g_v[sl]
                return 0

            lax.fori_loop(0, _V, pbody, 0)

            def seg_body(b, part):
                def abody(j, a):
                    sl = pl.ds(j * _L, _L)
                    return a + jnp.where(i_v[sl] == b, p_v[sl], 0.0)

                acc = lax.fori_loop(0, _V, abody,
                                    jnp.zeros((_L,), jnp.float32))
                return jnp.where(lanes == b, lane_sum(acc), part)

            part_v[...] = lax.fori_loop(first, last + 1, seg_body,
                                        jnp.zeros((_L,), jnp.float32))

        pltpu.sync_copy(part_v, acc_sh.at[pl.ds(sid * _B, _B)])

    plsc.subcore_barrier()

    @pl.when((cid == 0) & (sid == 0))
    def _():
        pltpu.sync_copy(acc_sh, all_v)
        tot = all_v[pl.ds(0, _B)]
        for t in range(1, _NS):
            tot = tot + all_v[pl.ds(t * _B, _B)]
        part_v[...] = tot
        pltpu.sync_copy(part_v, out_hbm)


def kernel(scores, gating, segment_ids):
    return _segsum_sc(scores, gating, segment_ids.astype(jnp.int32))


# speedup vs baseline: 1.0007x; 1.0007x over previous
import functools

import jax
import jax.numpy as jnp
from jax import lax
from jax.experimental import pallas as pl
from jax.experimental.pallas import tpu as pltpu
from jax.experimental.pallas import tpu_sc as plsc

_B = 16
_N = 32768
_NC = 2
_NS = 16
_L = 16
_NW = _NC * _NS
_C = _N // _NW
_V = _C // _L

_mesh = plsc.VectorSubcoreMesh(core_axis_name="c", subcore_axis_name="s")


@functools.partial(
    pl.kernel,
    mesh=_mesh,
    out_type=jax.ShapeDtypeStruct((_NC * _B,), jnp.float32),
    scratch_types=[
        pltpu.VMEM((_C,), jnp.float32),
        pltpu.VMEM((_C,), jnp.float32),
        pltpu.VMEM((_C,), jnp.int32),
        pltpu.VMEM((_B,), jnp.float32),
        pltpu.VMEM((_NS * _B,), jnp.float32),
        pltpu.VMEM_SHARED((_NW * _B,), jnp.float32),
        pltpu.SemaphoreType.DMA,
    ],
)
def _segsum_sc(scores_hbm, gating_hbm, ids_hbm, out_hbm,
               s_v, g_v, i_v, part_v, all_v, acc_sh, sem):
    cid = lax.axis_index("c")
    sid = lax.axis_index("s")
    wid = sid * _NC + cid
    base = wid * _C

    c1 = pltpu.async_copy(scores_hbm.at[pl.ds(base, _C)], s_v, sem)
    c2 = pltpu.async_copy(gating_hbm.at[pl.ds(base, _C)], g_v, sem)
    c3 = pltpu.async_copy(ids_hbm.at[pl.ds(base, _C)], i_v, sem)
    c1.wait()
    c2.wait()

    accs4 = [jnp.zeros((_L,), jnp.float32) for _ in range(4)]
    for j in range(_V):
        sl = pl.ds(j * _L, _L)
        accs4[j % 4] = accs4[j % 4] + s_v[sl] * g_v[sl]
    acc_all = (accs4[0] + accs4[1]) + (accs4[2] + accs4[3])

    c3.wait()
    first = i_v[pl.ds(0, _L)][0]
    last = i_v[pl.ds(_C - _L, _L)][_L - 1]
    lanes = lax.iota(jnp.int32, _L)

    def lane_sum(acc):
        half = [acc[2 * l] + acc[2 * l + 1] for l in range(_L // 2)]
        while len(half) > 1:
            half = [half[2 * l] + half[2 * l + 1]
                    for l in range(len(half) // 2)]
        return half[0]

    @pl.when(first == last)
    def _():
        part_v[...] = jnp.where(lanes == first, lane_sum(acc_all), 0.0)

    @pl.when(first != last)
    def _():
        def seg_body(b, part):
            def abody(j, a):
                sl = pl.ds(j * _L, _L)
                return a + jnp.where(i_v[sl] == b, s_v[sl] * g_v[sl], 0.0)

            acc = lax.fori_loop(0, _V, abody, jnp.zeros((_L,), jnp.float32))
            return jnp.where(lanes == b, lane_sum(acc), part)

        part_v[...] = lax.fori_loop(first, last + 1, seg_body,
                                    jnp.zeros((_L,), jnp.float32))

    pltpu.sync_copy(part_v, acc_sh.at[pl.ds((cid * _NS + sid) * _B, _B)])
    plsc.subcore_barrier()

    @pl.when(sid == 0)
    def _():
        pltpu.sync_copy(acc_sh.at[pl.ds(cid * _NS * _B, _NS * _B)], all_v)
        tot = all_v[pl.ds(0, _B)]
        for t in range(1, _NS):
            tot = tot + all_v[pl.ds(t * _B, _B)]
        part_v[...] = tot
        pltpu.sync_copy(part_v, out_hbm.at[pl.ds(cid * _B, _B)])


def kernel(scores, gating, segment_ids):
    partials = _segsum_sc(scores, gating, segment_ids.astype(jnp.int32))
    return partials[:_B] + partials[_B:]


# speedup vs baseline: 1.0358x; 1.0350x over previous
import functools

import jax
import jax.numpy as jnp
from jax import lax
from jax.experimental import pallas as pl
from jax.experimental.pallas import tpu as pltpu
from jax.experimental.pallas import tpu_sc as plsc

_B = 16
_N = 32768
_NC = 2
_NS = 16
_L = 16
_NW = _NC * _NS
_C = _N // _NW
_V = _C // _L

_mesh = plsc.VectorSubcoreMesh(core_axis_name="c", subcore_axis_name="s")


@functools.partial(
    pl.kernel,
    mesh=_mesh,
    out_type=jax.ShapeDtypeStruct((_NW * _B,), jnp.float32),
    scratch_types=[
        pltpu.VMEM((_C,), jnp.float32),
        pltpu.VMEM((_C,), jnp.float32),
        pltpu.VMEM((_C,), jnp.int32),
        pltpu.VMEM((_B,), jnp.float32),
        pltpu.SemaphoreType.DMA,
    ],
)
def _segsum_sc(scores_hbm, gating_hbm, ids_hbm, out_hbm,
               s_v, g_v, i_v, part_v, sem):
    cid = lax.axis_index("c")
    sid = lax.axis_index("s")
    wid = sid * _NC + cid
    base = wid * _C

    c1 = pltpu.async_copy(scores_hbm.at[pl.ds(base, _C)], s_v, sem)
    c2 = pltpu.async_copy(gating_hbm.at[pl.ds(base, _C)], g_v, sem)
    c3 = pltpu.async_copy(ids_hbm.at[pl.ds(base, _C)], i_v, sem)
    c1.wait()
    c2.wait()
    c3.wait()

    first = i_v[pl.ds(0, _L)][0]
    last = i_v[pl.ds(_C - _L, _L)][_L - 1]
    lanes = lax.iota(jnp.int32, _L)

    def lane_sum(acc):
        half = [acc[2 * l] + acc[2 * l + 1] for l in range(_L // 2)]
        while len(half) > 1:
            half = [half[2 * l] + half[2 * l + 1]
                    for l in range(len(half) // 2)]
        return half[0]

    @pl.when(first == last)
    def _():
        def fbody(j, accs):
            j4 = j * 4
            new = []
            for k in range(4):
                sl = pl.ds((j4 + k) * _L, _L)
                new.append(accs[k] + s_v[sl] * g_v[sl])
            return tuple(new)

        z = jnp.zeros((_L,), jnp.float32)
        a0, a1, a2, a3 = lax.fori_loop(0, _V // 4, fbody, (z, z, z, z))
        acc = (a0 + a1) + (a2 + a3)
        part_v[...] = jnp.where(lanes == first, lane_sum(acc), 0.0)

    @pl.when(first != last)
    def _():
        def seg_body(b, part):
            def abody(j, a):
                sl = pl.ds(j * _L, _L)
                return a + jnp.where(i_v[sl] == b, s_v[sl] * g_v[sl], 0.0)

            acc = lax.fori_loop(0, _V, abody, jnp.zeros((_L,), jnp.float32))
            return jnp.where(lanes == b, lane_sum(acc), part)

        part_v[...] = lax.fori_loop(first, last + 1, seg_body,
                                    jnp.zeros((_L,), jnp.float32))

    pltpu.sync_copy(part_v, out_hbm.at[pl.ds(wid * _B, _B)])


def kernel(scores, gating, segment_ids):
    partials = _segsum_sc(scores, gating, segment_ids.astype(jnp.int32))
    return jnp.sum(partials.reshape(_NW, _B), axis=0)


# speedup vs baseline: 1.0446x; 1.0085x over previous
import functools

import jax
import jax.numpy as jnp
from jax import lax
from jax.experimental import pallas as pl
from jax.experimental.pallas import tpu as pltpu
from jax.experimental.pallas import tpu_sc as plsc

_B = 16
_N = 32768
_NC = 2
_NS = 16
_L = 16
_NW = _NC * _NS
_C = _N // _NW
_V = _C // _L

_mesh = plsc.VectorSubcoreMesh(core_axis_name="c", subcore_axis_name="s")


@functools.partial(
    pl.kernel,
    mesh=_mesh,
    out_type=jax.ShapeDtypeStruct((_NW * _B,), jnp.float32),
    scratch_types=[
        pltpu.VMEM((_C,), jnp.float32),
        pltpu.VMEM((_C,), jnp.float32),
        pltpu.VMEM((_C,), jnp.int32),
        pltpu.VMEM((_B,), jnp.float32),
        pltpu.SemaphoreType.DMA,
    ],
)
def _segsum_sc(scores_hbm, gating_hbm, ids_hbm, out_hbm,
               s_v, g_v, i_v, part_v, sem):
    cid = lax.axis_index("c")
    sid = lax.axis_index("s")
    wid = sid * _NC + cid
    base = wid * _C

    c1 = pltpu.async_copy(scores_hbm.at[pl.ds(base, _C)], s_v, sem)
    c2 = pltpu.async_copy(gating_hbm.at[pl.ds(base, _C)], g_v, sem)
    c3 = pltpu.async_copy(ids_hbm.at[pl.ds(base, _C)], i_v, sem)
    c1.wait()
    c2.wait()
    c3.wait()

    first = i_v[pl.ds(0, _L)][0]
    last = i_v[pl.ds(_C - _L, _L)][_L - 1]
    lanes = lax.iota(jnp.int32, _L)

    def lane_sum(acc):
        half = [acc[2 * l] + acc[2 * l + 1] for l in range(_L // 2)]
        while len(half) > 1:
            half = [half[2 * l] + half[2 * l + 1]
                    for l in range(len(half) // 2)]
        return half[0]

    def seg_body(b, part):
        def abody(j, a):
            sl = pl.ds(j * _L, _L)
            return a + jnp.where(i_v[sl] == b, s_v[sl] * g_v[sl], 0.0)

        acc = lax.fori_loop(0, _V, abody, jnp.zeros((_L,), jnp.float32))
        return jnp.where(lanes == b, lane_sum(acc), part)

    part_v[...] = lax.fori_loop(first, last + 1, seg_body,
                                jnp.zeros((_L,), jnp.float32))

    pltpu.sync_copy(part_v, out_hbm.at[pl.ds(wid * _B, _B)])


def kernel(scores, gating, segment_ids):
    partials = _segsum_sc(scores, gating, segment_ids.astype(jnp.int32))
    return jnp.sum(partials.reshape(_NW, _B), axis=0)


# speedup vs baseline: 1.0575x; 1.0123x over previous
import functools

import jax
import jax.numpy as jnp
from jax import lax
from jax.experimental import pallas as pl
from jax.experimental.pallas import tpu as pltpu
from jax.experimental.pallas import tpu_sc as plsc

_B = 16
_N = 32768
_NC = 2
_NS = 16
_L = 16
_NW = _NC * _NS
_C = _N // _NW
_V = _C // _L

_mesh = plsc.VectorSubcoreMesh(core_axis_name="c", subcore_axis_name="s")


@functools.partial(
    pl.kernel,
    mesh=_mesh,
    out_type=jax.ShapeDtypeStruct((_NW * _B,), jnp.float32),
    scratch_types=[
        pltpu.VMEM((_C,), jnp.float32),
        pltpu.VMEM((_C,), jnp.float32),
        pltpu.VMEM((_C,), jnp.int32),
        pltpu.VMEM((_B,), jnp.float32),
        pltpu.SemaphoreType.DMA,
    ],
)
def _segsum_sc(scores_hbm, gating_hbm, ids_hbm, out_hbm,
               s_v, g_v, i_v, part_v, sem):
    cid = lax.axis_index("c")
    sid = lax.axis_index("s")
    wid = sid * _NC + cid
    base = wid * _C

    c1 = pltpu.async_copy(scores_hbm.at[pl.ds(base, _C)], s_v, sem)
    c2 = pltpu.async_copy(gating_hbm.at[pl.ds(base, _C)], g_v, sem)
    c3 = pltpu.async_copy(ids_hbm.at[pl.ds(base, _C)], i_v, sem)
    c1.wait()
    c2.wait()
    c3.wait()

    first = i_v[pl.ds(0, _L)][0]
    last = i_v[pl.ds(_C - _L, _L)][_L - 1]
    lanes = lax.iota(jnp.int32, _L)

    def lane_sum(acc):
        half = [acc[2 * l] + acc[2 * l + 1] for l in range(_L // 2)]
        while len(half) > 1:
            half = [half[2 * l] + half[2 * l + 1]
                    for l in range(len(half) // 2)]
        return half[0]

    def seg_body(b, part):
        def abody(j, accs):
            a0, a1 = accs
            sl0 = pl.ds((2 * j) * _L, _L)
            sl1 = pl.ds((2 * j + 1) * _L, _L)
            a0 = a0 + jnp.where(i_v[sl0] == b, s_v[sl0] * g_v[sl0], 0.0)
            a1 = a1 + jnp.where(i_v[sl1] == b, s_v[sl1] * g_v[sl1], 0.0)
            return (a0, a1)

        z = jnp.zeros((_L,), jnp.float32)
        a0, a1 = lax.fori_loop(0, _V // 2, abody, (z, z))
        return jnp.where(lanes == b, lane_sum(a0 + a1), part)

    part_v[...] = lax.fori_loop(first, last + 1, seg_body,
                                jnp.zeros((_L,), jnp.float32))

    pltpu.sync_copy(part_v, out_hbm.at[pl.ds(wid * _B, _B)])


def kernel(scores, gating, segment_ids):
    partials = _segsum_sc(scores, gating, segment_ids.astype(jnp.int32))
    return jnp.sum(partials.reshape(_NW, _B), axis=0)


# speedup vs baseline: 1.0617x; 1.0040x over previous
import functools

import jax
import jax.numpy as jnp
from jax import lax
from jax.experimental import pallas as pl
from jax.experimental.pallas import tpu as pltpu
from jax.experimental.pallas import tpu_sc as plsc

_B = 16
_N = 32768
_NC = 2
_NS = 16
_L = 16
_NW = _NC * _NS
_C = _N // _NW
_V = _C // _L

_mesh = plsc.VectorSubcoreMesh(core_axis_name="c", subcore_axis_name="s")


@functools.partial(
    pl.kernel,
    mesh=_mesh,
    out_type=jax.ShapeDtypeStruct((_NW * _B,), jnp.float32),
    scratch_types=[
        pltpu.VMEM((_C,), jnp.float32),
        pltpu.VMEM((_C,), jnp.float32),
        pltpu.VMEM((_C,), jnp.int32),
        pltpu.VMEM((_B,), jnp.float32),
        pltpu.SemaphoreType.DMA,
    ],
)
def _segsum_sc(scores_hbm, gating_hbm, ids_hbm, out_hbm,
               s_v, g_v, i_v, part_v, sem):
    cid = lax.axis_index("c")
    sid = lax.axis_index("s")
    wid = sid * _NC + cid
    base = wid * _C

    c1 = pltpu.async_copy(scores_hbm.at[pl.ds(base, _C)], s_v, sem)
    c2 = pltpu.async_copy(gating_hbm.at[pl.ds(base, _C)], g_v, sem)
    c3 = pltpu.async_copy(ids_hbm.at[pl.ds(base, _C)], i_v, sem)
    c1.wait()
    c2.wait()
    c3.wait()

    first = i_v[pl.ds(0, _L)][0]
    last = i_v[pl.ds(_C - _L, _L)][_L - 1]
    lanes = lax.iota(jnp.int32, _L)

    def lane_sum(acc):
        half = [acc[2 * l] + acc[2 * l + 1] for l in range(_L // 2)]
        while len(half) > 1:
            half = [half[2 * l] + half[2 * l + 1]
                    for l in range(len(half) // 2)]
        return half[0]

    def seg_body(b, part):
        def abody(j, accs):
            new = []
            for k in range(4):
                sl = pl.ds((4 * j + k) * _L, _L)
                new.append(accs[k]
                           + jnp.where(i_v[sl] == b, s_v[sl] * g_v[sl], 0.0))
            return tuple(new)

        z = jnp.zeros((_L,), jnp.float32)
        a0, a1, a2, a3 = lax.fori_loop(0, _V // 4, abody, (z, z, z, z))
        return jnp.where(lanes == b, lane_sum((a0 + a1) + (a2 + a3)), part)

    part_v[...] = lax.fori_loop(first, last + 1, seg_body,
                                jnp.zeros((_L,), jnp.float32))

    pltpu.sync_copy(part_v, out_hbm.at[pl.ds(wid * _B, _B)])


def kernel(scores, gating, segment_ids):
    partials = _segsum_sc(scores, gating, segment_ids.astype(jnp.int32))
    return jnp.sum(partials.reshape(_NW, _B), axis=0)
